# baseline (speedup 1.0000x reference)
import jax
import jax.numpy as jnp
from jax import lax
from jax.experimental import pallas as pl
from jax.experimental.pallas import tpu as pltpu
from jax.experimental.pallas import tpu_sc as plsc

N = 10000
E = 320000
D = 128
NC = 2
NS = 16
NW = NC * NS
CHUNK = 80
RCH = 125
EPW = RCH * CHUNK
N_HIST = 10240
N_ACC = 10112
RPT_H = N_HIST // NS
RPT_A = N_ACC // NS

_mesh = plsc.VectorSubcoreMesh(core_axis_name="c", subcore_axis_name="s")
_MASK16 = 0xFFFF


def _unpack_src(pk_v, j, out_ref):
    for i in range(CHUNK // 16):
        pv = pk_v[pl.ds(CHUNK * j + i * 16, 16)]
        out_ref[pl.ds(i * 16, 16)] = pv & _MASK16


def _unpack_dst(pk_v, j, out_ref):
    for i in range(CHUNK // 16):
        pv = pk_v[pl.ds(CHUNK * j + i * 16, 16)]
        out_ref[pl.ds(i * 16, 16)] = lax.shift_right_logical(pv, 16)


_DEG_BATCH = 5


def _deg_body(src1_hbm, dst1_hbm, ones1_hbm, zeros1_hbm, deg_out, pk_out,
              src_v, dst_v, ones_v, didx, hist, sem):
    c = lax.axis_index("c")
    s = lax.axis_index("s")
    w = c * NS + s
    pltpu.sync_copy(zeros1_hbm, hist.at[pl.ds(s * RPT_H, RPT_H)])
    plsc.subcore_barrier()
    pltpu.sync_copy(src1_hbm.at[pl.ds(w * EPW, EPW)], src_v)
    pltpu.sync_copy(dst1_hbm.at[pl.ds(w * EPW, EPW)], dst_v)
    pltpu.sync_copy(ones1_hbm, ones_v)

    def batch(kb, carry):
        j0 = kb * _DEG_BATCH
        for i in range(_DEG_BATCH):
            for q in range(CHUNK // 16):
                didx[i][pl.ds(q * 16, 16)] = dst_v[
                    pl.ds(CHUNK * (j0 + i) + q * 16, 16)]
            pltpu.async_copy(ones_v, hist.at[didx[i]], sem, add=True)
        for i in range(_DEG_BATCH):
            for q in range(CHUNK // 16):
                sl = pl.ds(CHUNK * (j0 + i) + q * 16, 16)
                src_v[sl] = src_v[sl] | lax.shift_left(dst_v[sl], 16)
        for i in range(_DEG_BATCH):
            pltpu.make_async_copy(ones_v, hist.at[didx[i]], sem).wait()
        return carry

    lax.fori_loop(0, RCH // _DEG_BATCH, batch, 0)
    pltpu.sync_copy(src_v, pk_out.at[pl.ds(w * EPW, EPW)])
    plsc.subcore_barrier()
    pltpu.sync_copy(hist.at[pl.ds(s * RPT_H, RPT_H)],
                    deg_out.at[c, pl.ds(s * RPT_H, RPT_H)])


_deg_call = pl.kernel(
    _deg_body,
    out_type=(
        jax.ShapeDtypeStruct((NC, N_HIST), jnp.float32),
        jax.ShapeDtypeStruct((E,), jnp.int32),
    ),
    mesh=_mesh,
    scratch_types=[
        pltpu.VMEM((EPW,), jnp.int32),
        pltpu.VMEM((EPW,), jnp.int32),
        pltpu.VMEM((CHUNK,), jnp.float32),
        [pltpu.VMEM((CHUNK,), jnp.int32) for _ in range(_DEG_BATCH)],
        pltpu.VMEM_SHARED((N_HIST,), jnp.float32),
        pltpu.SemaphoreType.DMA,
    ],
)


def _agg_body(pk_hbm, xs_hbm, zrow_hbm, s_out,
              pk_v, sidx, didx, rows, acc, sems):
    c = lax.axis_index("c")
    s = lax.axis_index("s")
    w = c * NS + s
    base = s * RPT_A
    pltpu.sync_copy(zrow_hbm, rows[0])
    for k in range(RPT_A // CHUNK):
        pltpu.sync_copy(rows[0], acc.at[pl.ds(base + k * CHUNK, CHUNK)])
    tail = RPT_A - (RPT_A // CHUNK) * CHUNK
    pltpu.sync_copy(rows[0].at[pl.ds(0, tail)],
                    acc.at[pl.ds(base + RPT_A - tail, tail)])
    plsc.subcore_barrier()

    pltpu.sync_copy(pk_hbm.at[pl.ds(w * EPW, EPW)], pk_v)

    for j in range(2):
        _unpack_src(pk_v, j, sidx[j])
        pltpu.async_copy(xs_hbm.at[sidx[j]], rows[j], sems[j])

    def body(k, carry):
        for i in range(3):
            j = 3 * k + i
            nxt = (i + 2) % 3
            pltpu.make_async_copy(
                xs_hbm.at[sidx[i]], rows[i], sems[i]).wait()
            _unpack_src(pk_v, j + 2, sidx[nxt])
            pltpu.async_copy(xs_hbm.at[sidx[nxt]], rows[nxt], sems[nxt])
            _unpack_dst(pk_v, j, didx)
            pltpu.sync_copy(rows[i], acc.at[didx], add=True)
        return carry

    nloop = (RCH - 2) // 3
    lax.fori_loop(0, nloop, body, 0)
    for j in range(3 * nloop, RCH):
        i = j % 3
        pltpu.make_async_copy(xs_hbm.at[sidx[i]], rows[i], sems[i]).wait()
        _unpack_dst(pk_v, j, didx)
        pltpu.sync_copy(rows[i], acc.at[didx], add=True)

    plsc.subcore_barrier()
    pltpu.sync_copy(acc.at[pl.ds(base, RPT_A)],
                    s_out.at[c, pl.ds(base, RPT_A)])


_agg_call = pl.kernel(
    _agg_body,
    out_type=jax.ShapeDtypeStruct((NC, N_ACC, D), jnp.float32),
    mesh=_mesh,
    scratch_types=[
        pltpu.VMEM((EPW,), jnp.int32),
        [pltpu.VMEM((CHUNK,), jnp.int32) for _ in range(3)],
        pltpu.VMEM((CHUNK,), jnp.int32),
        [pltpu.VMEM((CHUNK, D), jnp.float32) for _ in range(3)],
        pltpu.VMEM_SHARED((N_ACC, D), jnp.float32),
        [pltpu.SemaphoreType.DMA for _ in range(3)],
    ],
)


def _scale_body(x_ref, d0_ref, d1_ref, xs_ref, dis_ref):
    deg = d0_ref[...] + d1_ref[...] + 1.0
    dis = lax.rsqrt(deg)
    dis_ref[...] = dis
    xs_ref[...] = dis * x_ref[...]


_B_BLK = 1024


def _scale_call(x, d_flat):
    nb = N_HIST // _B_BLK
    return pl.pallas_call(
        _scale_body,
        grid=(nb,),
        in_specs=[
            pl.BlockSpec((_B_BLK, D), lambda i: (i, 0)),
            pl.BlockSpec((_B_BLK, 1), lambda i: (i, 0)),
            pl.BlockSpec((_B_BLK, 1), lambda i: (i + nb, 0)),
        ],
        out_specs=[
            pl.BlockSpec((_B_BLK, D), lambda i: (i, 0)),
            pl.BlockSpec((_B_BLK, 1), lambda i: (i, 0)),
        ],
        out_shape=[
            jax.ShapeDtypeStruct((N, D), jnp.float32),
            jax.ShapeDtypeStruct((N_HIST, 1), jnp.float32),
        ],
    )(x, d_flat, d_flat)


def _final_body(s0_ref, s1_ref, xs_ref, dis_ref, w_ref, b_ref, nz_ref, o_ref):
    t = dis_ref[...] * (s0_ref[...] + s1_ref[...] + xs_ref[...])
    h = jnp.dot(t, w_ref[...], preferred_element_type=jnp.float32)
    o_ref[...] = jnp.maximum(h + b_ref[...], 0.0) + nz_ref[...]


_D_BLK = 632
_NB = N_ACC // _D_BLK


def _final_call(s_flat, xs, dis, W, b2, noise):
    return pl.pallas_call(
        _final_body,
        grid=(_NB,),
        in_specs=[
            pl.BlockSpec((_D_BLK, D), lambda i: (i, 0)),
            pl.BlockSpec((_D_BLK, D), lambda i: (i + _NB, 0)),
            pl.BlockSpec((_D_BLK, D), lambda i: (i, 0)),
            pl.BlockSpec((_D_BLK, 1), lambda i: (i, 0)),
            pl.BlockSpec((D, D), lambda i: (0, 0)),
            pl.BlockSpec((1, D), lambda i: (0, 0)),
            pl.BlockSpec((_D_BLK, D), lambda i: (i, 0)),
        ],
        out_specs=pl.BlockSpec((_D_BLK, D), lambda i: (i, 0)),
        out_shape=jax.ShapeDtypeStruct((N, D), jnp.float32),
    )(s_flat, s_flat, xs, dis, W, b2, noise)


def kernel(x, edge_index, W, b):
    ones1 = jnp.ones((CHUNK,), jnp.float32)
    zeros1 = jnp.zeros((RPT_H,), jnp.float32)
    zrow = jnp.zeros((CHUNK, D), jnp.float32)

    dego, packed = _deg_call(edge_index[0], edge_index[1], ones1, zeros1)
    xs, dis = _scale_call(x, dego.reshape(2 * N_HIST, 1))
    s_part = _agg_call(packed, xs, zrow)

    noise = jax.random.laplace(
        jax.random.fold_in(jax.random.key(42), 7), (N, D), jnp.float32)
    b2 = b.reshape(1, D)
    return _final_call(s_part.reshape(2 * N_ACC, D), xs, dis, W, b2, noise)


# speedup vs baseline: 1.0606x; 1.0606x over previous
import jax
import jax.numpy as jnp
from jax import lax
from jax.experimental import pallas as pl
from jax.experimental.pallas import tpu as pltpu
from jax.experimental.pallas import tpu_sc as plsc

N = 10000
E = 320000
D = 128
NC = 2
NS = 16
NW = NC * NS
CHUNK = 80
RCH = 125
EPW = RCH * CHUNK
N_HIST = 10240
N_ACC = 10112
RPT_H = N_HIST // NS
RPT_A = N_ACC // NS

_mesh = plsc.VectorSubcoreMesh(core_axis_name="c", subcore_axis_name="s")
_MASK16 = 0xFFFF


def _unpack_src(pk_v, j, out_ref):
    for i in range(CHUNK // 16):
        pv = pk_v[pl.ds(CHUNK * j + i * 16, 16)]
        out_ref[pl.ds(i * 16, 16)] = pv & _MASK16


def _unpack_dst(pk_v, j, out_ref):
    for i in range(CHUNK // 16):
        pv = pk_v[pl.ds(CHUNK * j + i * 16, 16)]
        out_ref[pl.ds(i * 16, 16)] = lax.shift_right_logical(pv, 16)


_DEG_BATCH = 5


def _deg_body(eif_hbm, ones1_hbm, zeros1_hbm, deg_out, pk_out,
              src_v, dst_v, ones_v, didx, hist, sem):
    c = lax.axis_index("c")
    s = lax.axis_index("s")
    w = c * NS + s
    pltpu.sync_copy(zeros1_hbm, hist.at[pl.ds(s * RPT_H, RPT_H)])
    plsc.subcore_barrier()
    pltpu.sync_copy(eif_hbm.at[pl.ds(w * EPW, EPW)], src_v)
    pltpu.sync_copy(eif_hbm.at[pl.ds(E + w * EPW, EPW)], dst_v)
    pltpu.sync_copy(ones1_hbm, ones_v)

    def batch(kb, carry):
        j0 = kb * _DEG_BATCH
        for i in range(_DEG_BATCH):
            for q in range(CHUNK // 16):
                didx[i][pl.ds(q * 16, 16)] = dst_v[
                    pl.ds(CHUNK * (j0 + i) + q * 16, 16)]
            pltpu.async_copy(ones_v, hist.at[didx[i]], sem, add=True)
        for i in range(_DEG_BATCH):
            for q in range(CHUNK // 16):
                sl = pl.ds(CHUNK * (j0 + i) + q * 16, 16)
                src_v[sl] = src_v[sl] | lax.shift_left(dst_v[sl], 16)
        for i in range(_DEG_BATCH):
            pltpu.make_async_copy(ones_v, hist.at[didx[i]], sem).wait()
        return carry

    lax.fori_loop(0, RCH // _DEG_BATCH, batch, 0)
    pltpu.sync_copy(src_v, pk_out.at[pl.ds(w * EPW, EPW)])
    plsc.subcore_barrier()
    pltpu.sync_copy(hist.at[pl.ds(s * RPT_H, RPT_H)],
                    deg_out.at[c, pl.ds(s * RPT_H, RPT_H)])


_deg_call = pl.kernel(
    _deg_body,
    out_type=(
        jax.ShapeDtypeStruct((NC, N_HIST), jnp.float32),
        jax.ShapeDtypeStruct((E,), jnp.int32),
    ),
    mesh=_mesh,
    scratch_types=[
        pltpu.VMEM((EPW,), jnp.int32),
        pltpu.VMEM((EPW,), jnp.int32),
        pltpu.VMEM((CHUNK,), jnp.float32),
        [pltpu.VMEM((CHUNK,), jnp.int32) for _ in range(_DEG_BATCH)],
        pltpu.VMEM_SHARED((N_HIST,), jnp.float32),
        pltpu.SemaphoreType.DMA,
    ],
)


def _agg_body(pk_hbm, xs_hbm, zrow_hbm, s_out,
              pk_v, sidx, didx, rows, acc, sems):
    c = lax.axis_index("c")
    s = lax.axis_index("s")
    w = c * NS + s
    base = s * RPT_A
    pltpu.sync_copy(zrow_hbm, rows[0])
    for k in range(RPT_A // CHUNK):
        pltpu.sync_copy(rows[0], acc.at[pl.ds(base + k * CHUNK, CHUNK)])
    tail = RPT_A - (RPT_A // CHUNK) * CHUNK
    pltpu.sync_copy(rows[0].at[pl.ds(0, tail)],
                    acc.at[pl.ds(base + RPT_A - tail, tail)])
    plsc.subcore_barrier()

    pltpu.sync_copy(pk_hbm.at[pl.ds(w * EPW, EPW)], pk_v)

    for j in range(2):
        _unpack_src(pk_v, j, sidx[j])
        pltpu.async_copy(xs_hbm.at[sidx[j]], rows[j], sems[j])

    def body(k, carry):
        for i in range(3):
            j = 3 * k + i
            nxt = (i + 2) % 3
            pltpu.make_async_copy(
                xs_hbm.at[sidx[i]], rows[i], sems[i]).wait()
            _unpack_src(pk_v, j + 2, sidx[nxt])
            pltpu.async_copy(xs_hbm.at[sidx[nxt]], rows[nxt], sems[nxt])
            _unpack_dst(pk_v, j, didx)
            pltpu.sync_copy(rows[i], acc.at[didx], add=True)
        return carry

    nloop = (RCH - 2) // 3
    lax.fori_loop(0, nloop, body, 0)
    for j in range(3 * nloop, RCH):
        i = j % 3
        pltpu.make_async_copy(xs_hbm.at[sidx[i]], rows[i], sems[i]).wait()
        _unpack_dst(pk_v, j, didx)
        pltpu.sync_copy(rows[i], acc.at[didx], add=True)

    plsc.subcore_barrier()
    pltpu.sync_copy(acc.at[pl.ds(base, RPT_A)],
                    s_out.at[c, pl.ds(base, RPT_A)])


_agg_call = pl.kernel(
    _agg_body,
    out_type=jax.ShapeDtypeStruct((NC, N_ACC, D), jnp.float32),
    mesh=_mesh,
    scratch_types=[
        pltpu.VMEM((EPW,), jnp.int32),
        [pltpu.VMEM((CHUNK,), jnp.int32) for _ in range(3)],
        pltpu.VMEM((CHUNK,), jnp.int32),
        [pltpu.VMEM((CHUNK, D), jnp.float32) for _ in range(3)],
        pltpu.VMEM_SHARED((N_ACC, D), jnp.float32),
        [pltpu.SemaphoreType.DMA for _ in range(3)],
    ],
)


def _scale_body(x_ref, d0_ref, d1_ref, xs_ref, dis_ref):
    deg = d0_ref[...] + d1_ref[...] + 1.0
    dis = lax.rsqrt(deg)
    dis_ref[...] = dis
    xs_ref[...] = dis * x_ref[...]


_B_BLK = 1024


def _scale_call(x, d_flat):
    nb = N_HIST // _B_BLK
    return pl.pallas_call(
        _scale_body,
        grid=(nb,),
        in_specs=[
            pl.BlockSpec((_B_BLK, D), lambda i: (i, 0)),
            pl.BlockSpec((_B_BLK, 1), lambda i: (i, 0)),
            pl.BlockSpec((_B_BLK, 1), lambda i: (i + nb, 0)),
        ],
        out_specs=[
            pl.BlockSpec((_B_BLK, D), lambda i: (i, 0)),
            pl.BlockSpec((_B_BLK, 1), lambda i: (i, 0)),
        ],
        out_shape=[
            jax.ShapeDtypeStruct((N, D), jnp.float32),
            jax.ShapeDtypeStruct((N_HIST, 1), jnp.float32),
        ],
    )(x, d_flat, d_flat)


def _final_body(s0_ref, s1_ref, xs_ref, dis_ref, w_ref, b_ref, nz_ref, o_ref):
    t = dis_ref[...] * (s0_ref[...] + s1_ref[...] + xs_ref[...])
    h = jnp.dot(t, w_ref[...], preferred_element_type=jnp.float32)
    o_ref[...] = jnp.maximum(h + b_ref[...], 0.0) + nz_ref[...]


_D_BLK = 632
_NB = N_ACC // _D_BLK


def _final_call(s_flat, xs, dis, W, b2, noise):
    return pl.pallas_call(
        _final_body,
        grid=(_NB,),
        in_specs=[
            pl.BlockSpec((_D_BLK, D), lambda i: (i, 0)),
            pl.BlockSpec((_D_BLK, D), lambda i: (i + _NB, 0)),
            pl.BlockSpec((_D_BLK, D), lambda i: (i, 0)),
            pl.BlockSpec((_D_BLK, 1), lambda i: (i, 0)),
            pl.BlockSpec((D, D), lambda i: (0, 0)),
            pl.BlockSpec((1, D), lambda i: (0, 0)),
            pl.BlockSpec((_D_BLK, D), lambda i: (i, 0)),
        ],
        out_specs=pl.BlockSpec((_D_BLK, D), lambda i: (i, 0)),
        out_shape=jax.ShapeDtypeStruct((N, D), jnp.float32),
    )(s_flat, s_flat, xs, dis, W, b2, noise)


def kernel(x, edge_index, W, b):
    ones1 = jnp.ones((CHUNK,), jnp.float32)
    zeros1 = jnp.zeros((RPT_H,), jnp.float32)
    zrow = jnp.zeros((CHUNK, D), jnp.float32)

    dego, packed = _deg_call(edge_index.reshape(2 * E), ones1, zeros1)
    xs, dis = _scale_call(x, dego.reshape(2 * N_HIST, 1))
    s_part = _agg_call(packed, xs, zrow)

    noise = jax.random.laplace(
        jax.random.fold_in(jax.random.key(42), 7), (N, D), jnp.float32)
    b2 = b.reshape(1, D)
    return _final_call(s_part.reshape(2 * N_ACC, D), xs, dis, W, b2, noise)


# speedup vs baseline: 1.0924x; 1.0300x over previous
import jax
import jax.numpy as jnp
from jax import lax
from jax.experimental import pallas as pl
from jax.experimental.pallas import tpu as pltpu
from jax.experimental.pallas import tpu_sc as plsc

N = 10000
E = 320000
D = 128
NC = 2
NS = 16
NW = NC * NS
CHUNK = 80
RCH = 125
EPW = RCH * CHUNK
N_HIST = 10240
N_ACC = 10112
RPT_H = N_HIST // NS
RPT_A = N_ACC // NS

_mesh = plsc.VectorSubcoreMesh(core_axis_name="c", subcore_axis_name="s")
_MASK16 = 0xFFFF


def _unpack_src(pk_v, j, out_ref):
    for i in range(CHUNK // 16):
        pv = pk_v[pl.ds(CHUNK * j + i * 16, 16)]
        out_ref[pl.ds(i * 16, 16)] = pv & _MASK16


def _unpack_dst(pk_v, j, out_ref):
    for i in range(CHUNK // 16):
        pv = pk_v[pl.ds(CHUNK * j + i * 16, 16)]
        out_ref[pl.ds(i * 16, 16)] = lax.shift_right_logical(pv, 16)


_DEG_BATCH = 5


def _deg_body(eif_hbm, ones1_hbm, zeros1_hbm, deg_out, pk_out,
              src_v, dst_v, ones_v, didx, hist, sem):
    c = lax.axis_index("c")
    s = lax.axis_index("s")
    w = c * NS + s
    pltpu.sync_copy(zeros1_hbm, hist.at[pl.ds(s * RPT_H, RPT_H)])
    plsc.subcore_barrier()
    pltpu.sync_copy(eif_hbm.at[pl.ds(w * EPW, EPW)], src_v)
    pltpu.sync_copy(eif_hbm.at[pl.ds(E + w * EPW, EPW)], dst_v)
    pltpu.sync_copy(ones1_hbm, ones_v)

    def batch(kb, carry):
        j0 = kb * _DEG_BATCH
        for i in range(_DEG_BATCH):
            for q in range(CHUNK // 16):
                didx[i][pl.ds(q * 16, 16)] = dst_v[
                    pl.ds(CHUNK * (j0 + i) + q * 16, 16)]
            pltpu.async_copy(ones_v, hist.at[didx[i]], sem, add=True)
        for i in range(_DEG_BATCH):
            for q in range(CHUNK // 16):
                sl = pl.ds(CHUNK * (j0 + i) + q * 16, 16)
                src_v[sl] = src_v[sl] | lax.shift_left(dst_v[sl], 16)
        for i in range(_DEG_BATCH):
            pltpu.make_async_copy(ones_v, hist.at[didx[i]], sem).wait()
        return carry

    lax.fori_loop(0, RCH // _DEG_BATCH, batch, 0)
    pltpu.sync_copy(src_v, pk_out.at[pl.ds(w * EPW, EPW)])
    plsc.subcore_barrier()
    pltpu.sync_copy(hist.at[pl.ds(s * RPT_H, RPT_H)],
                    deg_out.at[c, pl.ds(s * RPT_H, RPT_H)])


_deg_call = pl.kernel(
    _deg_body,
    out_type=(
        jax.ShapeDtypeStruct((NC, N_HIST), jnp.float32),
        jax.ShapeDtypeStruct((E,), jnp.int32),
    ),
    mesh=_mesh,
    scratch_types=[
        pltpu.VMEM((EPW,), jnp.int32),
        pltpu.VMEM((EPW,), jnp.int32),
        pltpu.VMEM((CHUNK,), jnp.float32),
        [pltpu.VMEM((CHUNK,), jnp.int32) for _ in range(_DEG_BATCH)],
        pltpu.VMEM_SHARED((N_HIST,), jnp.float32),
        pltpu.SemaphoreType.DMA,
    ],
)


def _agg_body(pk_hbm, xs_hbm, zrow_hbm, s_out,
              pk_v, sidx, didx, rows, acc, sems):
    c = lax.axis_index("c")
    s = lax.axis_index("s")
    w = c * NS + s
    base = s * RPT_A
    pltpu.sync_copy(zrow_hbm, rows[0])
    for k in range(RPT_A // CHUNK):
        pltpu.sync_copy(rows[0], acc.at[pl.ds(base + k * CHUNK, CHUNK)])
    tail = RPT_A - (RPT_A // CHUNK) * CHUNK
    pltpu.sync_copy(rows[0].at[pl.ds(0, tail)],
                    acc.at[pl.ds(base + RPT_A - tail, tail)])
    plsc.subcore_barrier()

    pltpu.sync_copy(pk_hbm.at[pl.ds(w * EPW, EPW)], pk_v)

    for j in range(2):
        _unpack_src(pk_v, j, sidx[j])
        pltpu.async_copy(xs_hbm.at[sidx[j]], rows[j], sems[j])

    def body(k, carry):
        for i in range(3):
            j = 3 * k + i
            nxt = (i + 2) % 3
            pltpu.make_async_copy(
                xs_hbm.at[sidx[i]], rows[i], sems[i]).wait()
            _unpack_src(pk_v, j + 2, sidx[nxt])
            pltpu.async_copy(xs_hbm.at[sidx[nxt]], rows[nxt], sems[nxt])
            _unpack_dst(pk_v, j, didx)
            pltpu.sync_copy(rows[i], acc.at[didx], add=True)
        return carry

    nloop = (RCH - 2) // 3
    lax.fori_loop(0, nloop, body, 0)
    for j in range(3 * nloop, RCH):
        i = j % 3
        pltpu.make_async_copy(xs_hbm.at[sidx[i]], rows[i], sems[i]).wait()
        _unpack_dst(pk_v, j, didx)
        pltpu.sync_copy(rows[i], acc.at[didx], add=True)

    plsc.subcore_barrier()
    pltpu.sync_copy(acc.at[pl.ds(base, RPT_A)],
                    s_out.at[c, pl.ds(base, RPT_A)])


_agg_call = pl.kernel(
    _agg_body,
    out_type=jax.ShapeDtypeStruct((NC, N_ACC, D), jnp.float32),
    mesh=_mesh,
    scratch_types=[
        pltpu.VMEM((EPW,), jnp.int32),
        [pltpu.VMEM((CHUNK,), jnp.int32) for _ in range(3)],
        pltpu.VMEM((CHUNK,), jnp.int32),
        [pltpu.VMEM((CHUNK, D), jnp.float32) for _ in range(3)],
        pltpu.VMEM_SHARED((N_ACC, D), jnp.float32),
        [pltpu.SemaphoreType.DMA for _ in range(3)],
    ],
)


def _scale_body(x_ref, d_ref, xs_ref, dis_ref):
    deg = d_ref[...].astype(jnp.float32) + 1.0
    dis = lax.rsqrt(deg)
    dis_ref[...] = dis.astype(jnp.bfloat16)
    xs_ref[...] = dis * x_ref[...]


_B_BLK = 1024


def _scale_call(x, dsum):
    nb = N_HIST // _B_BLK
    return pl.pallas_call(
        _scale_body,
        grid=(nb,),
        in_specs=[
            pl.BlockSpec((_B_BLK, D), lambda i: (i, 0)),
            pl.BlockSpec((_B_BLK, 1), lambda i: (i, 0)),
        ],
        out_specs=[
            pl.BlockSpec((_B_BLK, D), lambda i: (i, 0)),
            pl.BlockSpec((_B_BLK, 1), lambda i: (i, 0)),
        ],
        out_shape=[
            jax.ShapeDtypeStruct((N, D), jnp.float32),
            jax.ShapeDtypeStruct((N_HIST, 1), jnp.bfloat16),
        ],
    )(x, dsum)


def _final_body(s0_ref, s1_ref, xs_ref, dis_ref, w_ref, b_ref, nz_ref, o_ref):
    t = dis_ref[...].astype(jnp.float32) * (
        s0_ref[...] + s1_ref[...] + xs_ref[...])
    h = jnp.dot(t, w_ref[...], preferred_element_type=jnp.float32)
    o_ref[...] = jnp.maximum(h + b_ref[...], 0.0) + nz_ref[...]


_D_BLK = 632
_NB = N_ACC // _D_BLK


def _final_call(s_flat, xs, dis, W, b2, noise):
    return pl.pallas_call(
        _final_body,
        grid=(_NB,),
        in_specs=[
            pl.BlockSpec((_D_BLK, D), lambda i: (i, 0)),
            pl.BlockSpec((_D_BLK, D), lambda i: (i + _NB, 0)),
            pl.BlockSpec((_D_BLK, D), lambda i: (i, 0)),
            pl.BlockSpec((_D_BLK, 1), lambda i: (i, 0)),
            pl.BlockSpec((D, D), lambda i: (0, 0)),
            pl.BlockSpec((1, D), lambda i: (0, 0)),
            pl.BlockSpec((_D_BLK, D), lambda i: (i, 0)),
        ],
        out_specs=pl.BlockSpec((_D_BLK, D), lambda i: (i, 0)),
        out_shape=jax.ShapeDtypeStruct((N, D), jnp.float32),
    )(s_flat, s_flat, xs, dis, W, b2, noise)


def kernel(x, edge_index, W, b):
    ones1 = jnp.ones((CHUNK,), jnp.float32)
    zeros1 = jnp.zeros((RPT_H,), jnp.float32)
    zrow = jnp.zeros((CHUNK, D), jnp.float32)

    dego, packed = _deg_call(edge_index.reshape(2 * E), ones1, zeros1)
    dsum = (dego[0] + dego[1]).astype(jnp.bfloat16).reshape(N_HIST, 1)
    xs, dis = _scale_call(x, dsum)
    s_part = _agg_call(packed, xs, zrow)

    noise = jax.random.laplace(
        jax.random.fold_in(jax.random.key(42), 7), (N, D), jnp.float32)
    b2 = b.reshape(1, D)
    return _final_call(s_part.reshape(2 * N_ACC, D), xs, dis, W, b2, noise)
